# baseline (speedup 1.0000x reference)
import functools

import jax
import jax.numpy as jnp
from jax import lax
from jax.experimental import pallas as pl
from jax.experimental.pallas import tpu as pltpu
from jax.experimental.pallas import tpu_sc as plsc

DIM = 16
ALPHA = 0.001
LANES = 16
IDX_CHUNK = 128


def _dot_kernel(n_batch, n_workers, ij_hbm, p_hbm, m_hbm, out_hbm,
                ij_v, idx_i, idx_j, p_rows, m_rows, out_v, sem_p, sem_m):
    bpw = n_batch // n_workers
    wid = lax.axis_index("s") * 2 + lax.axis_index("c")
    base = wid * bpw

    pltpu.sync_copy(ij_hbm.at[pl.ds(base, bpw)], ij_v)

    def unzip_block(blk, _):
        b0 = blk * LANES
        rows = b0 + lax.iota(jnp.int32, LANES)
        col0 = jnp.zeros((LANES,), jnp.int32)
        col1 = jnp.ones((LANES,), jnp.int32)
        idx_i[pl.ds(b0, LANES)] = plsc.load_gather(ij_v, [rows, col0])
        idx_j[pl.ds(b0, LANES)] = plsc.load_gather(ij_v, [rows, col1])
        return 0

    lax.fori_loop(0, bpw // LANES, unzip_block, 0, unroll=4)

    n_chunks = bpw // IDX_CHUNK
    copies = []
    for c in range(n_chunks):
        s = pl.ds(c * IDX_CHUNK, IDX_CHUNK)
        copies.append(pltpu.async_copy(p_hbm.at[idx_i.at[s]], p_rows.at[s], sem_p))
        copies.append(pltpu.async_copy(m_hbm.at[idx_j.at[s]], m_rows.at[s], sem_m))
    for cp in copies:
        cp.wait()

    def dot_block(blk, _):
        b0 = blk * LANES
        rows = b0 + lax.iota(jnp.int32, LANES)
        acc = jnp.zeros((LANES,), jnp.float32)
        for d in range(DIM):
            col = jnp.full((LANES,), d, jnp.int32)
            pv = plsc.load_gather(p_rows, [rows, col])
            mv = plsc.load_gather(m_rows, [rows, col])
            acc = acc + pv * mv
        out_v[pl.ds(b0, LANES)] = acc * jnp.float32(ALPHA)
        return 0

    lax.fori_loop(0, bpw // LANES, dot_block, 0)

    pltpu.sync_copy(out_v, out_hbm.at[pl.ds(base, bpw)])


def kernel(ij, P, M):
    ij = ij.astype(jnp.int32)
    n_batch = ij.shape[0]
    info = plsc.get_sparse_core_info()
    n_workers = info.num_cores * info.num_subcores
    bpw = n_batch // n_workers

    mesh = plsc.VectorSubcoreMesh(core_axis_name="c", subcore_axis_name="s")
    run = pl.kernel(
        functools.partial(_dot_kernel, n_batch, n_workers),
        out_type=jax.ShapeDtypeStruct((n_batch,), jnp.float32),
        mesh=mesh,
        scratch_types=[
            pltpu.VMEM((bpw, 2), jnp.int32),
            pltpu.VMEM((bpw,), jnp.int32),
            pltpu.VMEM((bpw,), jnp.int32),
            pltpu.VMEM((bpw, DIM), jnp.float32),
            pltpu.VMEM((bpw, DIM), jnp.float32),
            pltpu.VMEM((bpw,), jnp.float32),
            pltpu.SemaphoreType.DMA,
            pltpu.SemaphoreType.DMA,
        ],
        compiler_params=pltpu.CompilerParams(
            needs_layout_passes=False, use_tc_tiling_on_sc=False
        ),
    )
    return run(ij, P, M)
